# baseline (speedup 1.0000x reference)
import jax
import jax.numpy as jnp
from jax import lax
from jax.experimental import pallas as pl

N, D, E, OUT = 2048, 1024, 8, 128
EPS = 2.220446049250313e-16
TBLK = 256
NEG_INF = float("-inf")


def _moe_dense_body(x_ref, wg_ref, w_ref, b_ref, o_ref):
    xb = x_ref[...]
    logits = jnp.dot(xb, wg_ref[...], preferred_element_type=jnp.float32)
    iota_e = lax.broadcasted_iota(jnp.int32, (TBLK, E), 1)
    m1 = jnp.max(logits, axis=1, keepdims=True)
    e0 = jnp.min(jnp.where(logits == m1, iota_e, E), axis=1, keepdims=True)
    masked = jnp.where(iota_e == e0, NEG_INF, logits)
    m2 = jnp.max(masked, axis=1, keepdims=True)
    e1 = jnp.min(jnp.where(masked == m2, iota_e, E), axis=1, keepdims=True)
    t = jnp.exp(m2 - m1)
    g0 = 1.0 / (1.0 + t)
    g1 = t / (1.0 + t)
    acc = jnp.zeros((TBLK, OUT), jnp.float32)
    for e in range(E):
        o = jnp.dot(xb, w_ref[e], preferred_element_type=jnp.float32)
        o = o + b_ref[e:e + 1, :]
        ge = jnp.where(e0 == e, g0, jnp.where(e1 == e, g1, 0.0))
        acc = acc + ge * jnp.exp(o)
    acc = jnp.where(acc == 0.0, EPS, acc)
    o_ref[...] = jnp.log(acc)


def kernel(x, w_gate, W_exp, b_exp):
    xb = x.astype(jnp.bfloat16)
    wg = w_gate.astype(jnp.bfloat16)
    W_bf = W_exp.astype(jnp.bfloat16)
    return pl.pallas_call(
        _moe_dense_body,
        grid=(N // TBLK,),
        in_specs=[
            pl.BlockSpec((TBLK, D), lambda i: (i, 0)),
            pl.BlockSpec((D, E), lambda i: (0, 0)),
            pl.BlockSpec((E, D, OUT), lambda i: (0, 0, 0)),
            pl.BlockSpec((E, OUT), lambda i: (0, 0)),
        ],
        out_specs=pl.BlockSpec((TBLK, OUT), lambda i: (i, 0)),
        out_shape=jax.ShapeDtypeStruct((N, OUT), jnp.float32),
    )(xb, wg, W_bf, b_exp)


# speedup vs baseline: 1.4795x; 1.4795x over previous
import jax
import jax.numpy as jnp
from jax import lax
from jax.experimental import pallas as pl
from jax.experimental.pallas import tpu as pltpu

N, D, E, OUT = 2048, 1024, 8, 128
EPS = 2.220446049250313e-16
TBLK = 256
NEG_INF = float("-inf")


def _moe_dense_body(x_ref, wg_ref, w_ref, b_ref, o_ref, wbf_ref):
    @pl.when(pl.program_id(0) == 0)
    def _():
        wbf_ref[...] = w_ref[...].astype(jnp.bfloat16)

    x = x_ref[...]
    logits = jnp.dot(x, wg_ref[...], preferred_element_type=jnp.float32)
    iota_e = lax.broadcasted_iota(jnp.int32, (TBLK, E), 1)
    m1 = jnp.max(logits, axis=1, keepdims=True)
    e0 = jnp.min(jnp.where(logits == m1, iota_e, E), axis=1, keepdims=True)
    masked = jnp.where(iota_e == e0, NEG_INF, logits)
    m2 = jnp.max(masked, axis=1, keepdims=True)
    e1 = jnp.min(jnp.where(masked == m2, iota_e, E), axis=1, keepdims=True)
    t = jnp.exp(m2 - m1)
    g0 = 1.0 / (1.0 + t)
    g1 = t / (1.0 + t)
    xb = x.astype(jnp.bfloat16)
    acc = jnp.zeros((TBLK, OUT), jnp.float32)
    for e in range(E):
        o = jnp.dot(xb, wbf_ref[e], preferred_element_type=jnp.float32)
        o = o + b_ref[e:e + 1, :]
        ge = jnp.where(e0 == e, g0, jnp.where(e1 == e, g1, 0.0))
        acc = acc + ge * jnp.exp(o)
    acc = jnp.where(acc == 0.0, EPS, acc)
    o_ref[...] = jnp.log(acc)


def kernel(x, w_gate, W_exp, b_exp):
    return pl.pallas_call(
        _moe_dense_body,
        grid=(N // TBLK,),
        in_specs=[
            pl.BlockSpec((TBLK, D), lambda i: (i, 0)),
            pl.BlockSpec((D, E), lambda i: (0, 0)),
            pl.BlockSpec((E, D, OUT), lambda i: (0, 0, 0)),
            pl.BlockSpec((E, OUT), lambda i: (0, 0)),
        ],
        out_specs=pl.BlockSpec((TBLK, OUT), lambda i: (i, 0)),
        out_shape=jax.ShapeDtypeStruct((N, OUT), jnp.float32),
        scratch_shapes=[pltpu.VMEM((E, D, OUT), jnp.bfloat16)],
    )(x, w_gate, W_exp, b_exp)
